# baseline (speedup 1.0000x reference)
import jax
import jax.numpy as jnp
from jax.experimental import pallas as pl
from jax.experimental.pallas import tpu as pltpu

_B, _T, _C, _H, _E = 8, 4, 768, 2048, 8
_N = _B * _T
_HE = _H * _E
_CE = _C * _E
_BLK = 2048
_HB = _BLK // _E
_NBLK = _HE // _BLK


def _moe_body(x_ref, wg_ref, wfc_ref, w2_ref, o_ref, w_scr, qacc_scr):
    j = pl.program_id(0)

    @pl.when(j == 0)
    def _():
        gate = jnp.dot(x_ref[...], wg_ref[...],
                       preferred_element_type=jnp.float32)
        e_iota = jax.lax.broadcasted_iota(jnp.int32, (_N, _E), 1)
        i1 = jnp.argmax(gate, axis=-1)
        is1 = e_iota == i1[:, None]
        m1 = jnp.max(gate, axis=-1, keepdims=True)
        gate2 = jnp.where(is1, -jnp.inf, gate)
        i2 = jnp.argmax(gate2, axis=-1)
        is2 = e_iota == i2[:, None]
        m2 = jnp.max(gate2, axis=-1, keepdims=True)
        w_scr[...] = jnp.where(is1, m1, 0.0) + jnp.where(is2, m2, 0.0)
        qacc_scr[...] = jnp.zeros((_N, _E, _C), jnp.float32)

    h = jnp.dot(x_ref[...], wfc_ref[...],
                preferred_element_type=jnp.float32)
    hw = jax.nn.gelu(h, approximate=True)

    w = w_scr[...]
    scale = jnp.concatenate([w[:, e:e + 1] for e in range(_E)], axis=0)
    hwT = hw.T.reshape(_HB, _E, _N)
    big_lhs = jnp.concatenate(
        [hwT[:, e, :].T for e in range(_E)], axis=0) * scale

    big_q3 = jax.lax.dot_general(
        big_lhs, w2_ref[...], (((1,), (0,)), ((), ())),
        preferred_element_type=jnp.float32)

    m_e = jax.lax.broadcasted_iota(jnp.int32, (_N, _E, _C), 2) % _E
    q = qacc_scr[...]
    for e in range(_E):
        q = q + jnp.where(m_e == e, big_q3[e * _N:(e + 1) * _N], 0.0)
    qacc_scr[...] = q

    @pl.when(j == _NBLK - 1)
    def _():
        qacc = qacc_scr[...]
        bands = []
        for a in range(_E):
            sT = qacc[:, a, :].T
            red = jnp.sum(sT.reshape(_C // _E, _E, _N), axis=1)
            bands.append(red.T)
        o_ref[...] = jnp.concatenate(bands, axis=1)


def _moe(x2, W_gate, W_fc, W2, interpret=False):
    return pl.pallas_call(
        _moe_body,
        grid=(_NBLK,),
        in_specs=[
            pl.BlockSpec((_N, _C), lambda j: (0, 0)),
            pl.BlockSpec((_C, _E), lambda j: (0, 0)),
            pl.BlockSpec((_C, _BLK), lambda j: (0, j)),
            pl.BlockSpec((_HB, _E, _C), lambda j: (j, 0, 0)),
        ],
        out_specs=pl.BlockSpec((_N, _C), lambda j: (0, 0)),
        out_shape=jax.ShapeDtypeStruct((_N, _C), jnp.float32),
        scratch_shapes=[
            pltpu.VMEM((_N, _E), jnp.float32),
            pltpu.VMEM((_N, _E, _C), jnp.float32),
        ],
        compiler_params=pltpu.CompilerParams(
            dimension_semantics=("arbitrary",),
        ),
        interpret=interpret,
    )(x2, W_gate, W_fc, W2)


def kernel(x, W_fc, W_proj, W_gate):
    Bx, Tx, Cx = x.shape
    x2 = x.reshape(Bx * Tx, Cx)
    W2 = W_proj.reshape(_H, _E, _C)
    o = _moe(x2, W_gate, W_fc, W2)
    return o.reshape(Bx, Tx, Cx)


# speedup vs baseline: 1.0294x; 1.0294x over previous
import jax
import jax.numpy as jnp
from jax.experimental import pallas as pl
from jax.experimental.pallas import tpu as pltpu

_B, _T, _C, _H, _E = 8, 4, 768, 2048, 8
_N = _B * _T
_HE = _H * _E
_CE = _C * _E
_BLK = 2048
_HB = _BLK // _E
_NBLK = _HE // _BLK


def _moe_body(x_ref, wg_ref, wfc_ref, w2_ref, o_ref, w_scr, qacc_scr):
    j = pl.program_id(0)

    @pl.when(j == 0)
    def _():
        gate = jnp.dot(x_ref[...], wg_ref[...],
                       preferred_element_type=jnp.float32)
        e_iota = jax.lax.broadcasted_iota(jnp.int32, (_N, _E), 1)
        i1 = jnp.argmax(gate, axis=-1)
        is1 = e_iota == i1[:, None]
        m1 = jnp.max(gate, axis=-1, keepdims=True)
        gate2 = jnp.where(is1, -jnp.inf, gate)
        i2 = jnp.argmax(gate2, axis=-1)
        is2 = e_iota == i2[:, None]
        m2 = jnp.max(gate2, axis=-1, keepdims=True)
        w_scr[...] = jnp.where(is1, m1, 0.0) + jnp.where(is2, m2, 0.0)
        qacc_scr[...] = jnp.zeros((_N, _E, _C), jnp.float32)

    h = jnp.dot(x_ref[...], wfc_ref[...],
                preferred_element_type=jnp.float32)
    hw = jax.nn.gelu(h, approximate=True)

    w = w_scr[...]
    scale = jnp.concatenate([w[:, e:e + 1] for e in range(_E)], axis=0)
    hwT = hw.T.reshape(_HB, _E, _N)
    big_lhs = jnp.concatenate(
        [hwT[:, e, :].T for e in range(_E)], axis=0) * scale

    big_q3 = jax.lax.dot_general(
        big_lhs.astype(jnp.bfloat16), w2_ref[...].astype(jnp.bfloat16),
        (((1,), (0,)), ((), ())),
        preferred_element_type=jnp.float32)

    m_e = jax.lax.broadcasted_iota(jnp.int32, (_N, _E, _C), 2) % _E
    q = qacc_scr[...]
    for e in range(_E):
        q = q + jnp.where(m_e == e, big_q3[e * _N:(e + 1) * _N], 0.0)
    qacc_scr[...] = q

    @pl.when(j == _NBLK - 1)
    def _():
        qacc = qacc_scr[...]
        bands = []
        for a in range(_E):
            sT = qacc[:, a, :].T
            red = jnp.sum(sT.reshape(_C // _E, _E, _N), axis=1)
            bands.append(red.T)
        o_ref[...] = jnp.concatenate(bands, axis=1)


def _moe(x2, W_gate, W_fc, W2, interpret=False):
    return pl.pallas_call(
        _moe_body,
        grid=(_NBLK,),
        in_specs=[
            pl.BlockSpec((_N, _C), lambda j: (0, 0)),
            pl.BlockSpec((_C, _E), lambda j: (0, 0)),
            pl.BlockSpec((_C, _BLK), lambda j: (0, j)),
            pl.BlockSpec((_HB, _E, _C), lambda j: (j, 0, 0)),
        ],
        out_specs=pl.BlockSpec((_N, _C), lambda j: (0, 0)),
        out_shape=jax.ShapeDtypeStruct((_N, _C), jnp.float32),
        scratch_shapes=[
            pltpu.VMEM((_N, _E), jnp.float32),
            pltpu.VMEM((_N, _E, _C), jnp.float32),
        ],
        compiler_params=pltpu.CompilerParams(
            dimension_semantics=("arbitrary",),
        ),
        interpret=interpret,
    )(x2, W_gate, W_fc, W2)


def kernel(x, W_fc, W_proj, W_gate):
    Bx, Tx, Cx = x.shape
    x2 = x.reshape(Bx * Tx, Cx)
    W2 = W_proj.reshape(_H, _E, _C)
    o = _moe(x2, W_gate, W_fc, W2)
    return o.reshape(Bx, Tx, Cx)
